# baseline (speedup 1.0000x reference)
import functools

import jax
import jax.numpy as jnp
from jax import lax
from jax.experimental import pallas as pl
from jax.experimental.pallas import tpu as pltpu
from jax.experimental.pallas import tpu_sc as plsc

N = 10000
D = 128
DH = D // 2
E = 320000

NC = 2
NS = 16
NW = NC * NS
CHUNK = 128
CPT = 80
NBUF = 4
EPT = CPT * CHUNK
E_PAD = EPT * NW
N_PAD = 10240
RPT = N_PAD // NS


def _sc_segment_sum(x0, x1, dst_r, src_r, zrows, zcnt, ones_c):
  mesh = plsc.VectorSubcoreMesh(core_axis_name="c", subcore_axis_name="s")

  @functools.partial(
      pl.kernel,
      out_type=[
          jax.ShapeDtypeStruct((NC, 2, N_PAD, DH), jnp.float32),
          jax.ShapeDtypeStruct((NC, N_PAD, 16), jnp.float32),
      ],
      mesh=mesh,
      compiler_params=pltpu.CompilerParams(use_tc_tiling_on_sc=False),
      scratch_types=[
          pltpu.VMEM((CPT, CHUNK), jnp.int32),
          pltpu.VMEM((CPT, CHUNK), jnp.int32),
          pltpu.VMEM((NBUF, CHUNK, DH), jnp.float32),
          pltpu.VMEM((CHUNK, 16), jnp.float32),
          pltpu.VMEM_SHARED((N_PAD, DH), jnp.float32),
          pltpu.VMEM_SHARED((N_PAD, 16), jnp.float32),
      ] + [pltpu.SemaphoreType.DMA] * NBUF,
  )
  def sc_kernel(x0_hbm, x1_hbm, dst_hbm, src_hbm, zr_hbm, zc_hbm, ones_hbm,
                sum_out, cnt_out,
                dst_v, src_v, rows_v, ones_v, acc_sh, cnt_sh, *sems):
    cid = lax.axis_index("c")
    sid = lax.axis_index("s")
    wid = sid * NC + cid
    r0 = sid * RPT

    pltpu.sync_copy(dst_hbm.at[wid], dst_v)
    pltpu.sync_copy(src_hbm.at[wid], src_v)
    pltpu.sync_copy(ones_hbm, ones_v)

    for h, xh_hbm in ((0, x0_hbm), (1, x1_hbm)):
      pltpu.sync_copy(zr_hbm, acc_sh.at[pl.ds(r0, RPT)])
      if h == 0:
        pltpu.sync_copy(zc_hbm, cnt_sh.at[pl.ds(r0, RPT)])
      plsc.subcore_barrier()

      def body(j, carry):
        pltpu.async_copy(xh_hbm.at[src_v.at[j]], rows_v.at[0], sems[0]).wait()
        pltpu.sync_copy(rows_v.at[0], acc_sh.at[dst_v.at[j]], add=True)
        if h == 0:
          pltpu.sync_copy(ones_v, cnt_sh.at[dst_v.at[j]], add=True)
        return carry

      lax.fori_loop(0, CPT, body, 0)
      plsc.subcore_barrier()

      pltpu.sync_copy(acc_sh.at[pl.ds(r0, RPT)],
                      sum_out.at[cid, h, pl.ds(r0, RPT)])
      if h == 0:
        pltpu.sync_copy(cnt_sh.at[pl.ds(r0, RPT)],
                        cnt_out.at[cid, pl.ds(r0, RPT)])
        plsc.subcore_barrier()

  return sc_kernel(x0, x1, dst_r, src_r, zrows, zcnt, ones_c)


def _tc_combine(x, psum, pcnt, ws_t, wn_t, bs, bn):
  blk = 1000

  def body(x_ref, p_ref, c_ref, ws_ref, wn_ref, bs_ref, bn_ref, o_ref):
    xb = x_ref[...]
    s = jnp.concatenate(
        [p_ref[0, 0] + p_ref[1, 0], p_ref[0, 1] + p_ref[1, 1]], axis=-1)
    ct = c_ref[0, :, 0] + c_ref[1, :, 0]
    mean = s / jnp.maximum(ct, 1.0)[:, None]
    nei = jnp.dot(mean, wn_ref[...], preferred_element_type=jnp.float32)
    nei = nei + bn_ref[...]
    sx = jnp.dot(xb, ws_ref[...], preferred_element_type=jnp.float32)
    sx = sx + bs_ref[...]
    o_ref[...] = jnp.maximum(sx + jnp.where(ct[:, None] > 0, nei, 0.0), 0.0)

  return pl.pallas_call(
      body,
      grid=(N // blk,),
      in_specs=[
          pl.BlockSpec((blk, D), lambda i: (i, 0)),
          pl.BlockSpec((NC, 2, blk, DH), lambda i: (0, 0, i, 0)),
          pl.BlockSpec((NC, blk, 16), lambda i: (0, i, 0)),
          pl.BlockSpec((D, D), lambda i: (0, 0)),
          pl.BlockSpec((D, D), lambda i: (0, 0)),
          pl.BlockSpec((1, D), lambda i: (0, 0)),
          pl.BlockSpec((1, D), lambda i: (0, 0)),
      ],
      out_specs=pl.BlockSpec((blk, D), lambda i: (i, 0)),
      out_shape=jax.ShapeDtypeStruct((N, D), jnp.float32),
  )(x, psum, pcnt, ws_t, wn_t, bs, bn)


def kernel(x, edge_index, W_self, b_self, W_nei, b_nei):
  dst = edge_index[0]
  src = edge_index[1]
  pad = E_PAD - E
  dst_p = jnp.concatenate([dst, jnp.full((pad,), N, dtype=jnp.int32)])
  src_p = jnp.concatenate([src, jnp.zeros((pad,), dtype=jnp.int32)])
  dst_r = dst_p.reshape(NW, CPT, CHUNK)
  src_r = src_p.reshape(NW, CPT, CHUNK)
  x0 = x[:, :DH]
  x1 = x[:, DH:]
  zrows = jnp.zeros((RPT, DH), jnp.float32)
  zcnt = jnp.zeros((RPT, 16), jnp.float32)
  ones_c = jnp.ones((CHUNK, 16), jnp.float32)
  psum, pcnt = _sc_segment_sum(x0, x1, dst_r, src_r, zrows, zcnt, ones_c)
  return _tc_combine(x, psum, pcnt, W_self.T, W_nei.T,
                     b_self[None, :], b_nei[None, :])


# speedup vs baseline: 2.1546x; 2.1546x over previous
import functools

import jax
import jax.numpy as jnp
from jax import lax
from jax.experimental import pallas as pl
from jax.experimental.pallas import tpu as pltpu
from jax.experimental.pallas import tpu_sc as plsc

N = 10000
D = 128
DH = D // 2
E = 320000

NC = 2
NS = 16
NW = NC * NS
CHUNK = 128
CPT = 80
NBUF = 4
EPT = CPT * CHUNK
E_PAD = EPT * NW
N_PAD = 10240
RPT = N_PAD // NS


def _sc_segment_sum(x0, x1, dst_r, src_r, zrows, zcnt, ones_c):
  mesh = plsc.VectorSubcoreMesh(core_axis_name="c", subcore_axis_name="s")

  @functools.partial(
      pl.kernel,
      out_type=[
          jax.ShapeDtypeStruct((NC, 2, N_PAD, DH), jnp.float32),
          jax.ShapeDtypeStruct((NC, N_PAD, 16), jnp.float32),
      ],
      mesh=mesh,
      compiler_params=pltpu.CompilerParams(use_tc_tiling_on_sc=False),
      scratch_types=[
          pltpu.VMEM((CPT, CHUNK), jnp.int32),
          pltpu.VMEM((CPT, CHUNK), jnp.int32),
          pltpu.VMEM((NBUF, CHUNK, DH), jnp.float32),
          pltpu.VMEM((CHUNK, 16), jnp.float32),
          pltpu.VMEM_SHARED((N_PAD, DH), jnp.float32),
          pltpu.VMEM_SHARED((N_PAD, 16), jnp.float32),
      ] + [pltpu.SemaphoreType.DMA] * NBUF,
  )
  def sc_kernel(x0_hbm, x1_hbm, dst_hbm, src_hbm, zr_hbm, zc_hbm, ones_hbm,
                sum_out, cnt_out,
                dst_v, src_v, rows_v, ones_v, acc_sh, cnt_sh, *sems):
    cid = lax.axis_index("c")
    sid = lax.axis_index("s")
    wid = sid * NC + cid
    r0 = sid * RPT

    pltpu.sync_copy(dst_hbm.at[wid], dst_v)
    pltpu.sync_copy(src_hbm.at[wid], src_v)
    pltpu.sync_copy(ones_hbm, ones_v)

    for h, xh_hbm in ((0, x0_hbm), (1, x1_hbm)):
      pltpu.sync_copy(zr_hbm, acc_sh.at[pl.ds(r0, RPT)])
      if h == 0:
        pltpu.sync_copy(zc_hbm, cnt_sh.at[pl.ds(r0, RPT)])
      plsc.subcore_barrier()

      def body(j, carry):
        pltpu.async_copy(xh_hbm.at[src_v.at[j]], rows_v.at[0], sems[0]).wait()
        pltpu.sync_copy(rows_v.at[0], acc_sh.at[dst_v.at[j]], add=True)
        if h == 0:
          pltpu.sync_copy(ones_v, cnt_sh.at[dst_v.at[j]], add=True)
        return carry

      lax.fori_loop(0, CPT, body, 0)
      plsc.subcore_barrier()

      pltpu.sync_copy(acc_sh.at[pl.ds(r0, RPT)],
                      sum_out.at[cid, h, pl.ds(r0, RPT)])
      if h == 0:
        pltpu.sync_copy(cnt_sh.at[pl.ds(r0, RPT)],
                        cnt_out.at[cid, pl.ds(r0, RPT)])
        plsc.subcore_barrier()

  return sc_kernel(x0, x1, dst_r, src_r, zrows, zcnt, ones_c)


def _tc_combine(x, psum, pcnt, ws_t, wn_t, bs, bn):
  blk = 1000

  def body(x_ref, p_ref, c_ref, ws_ref, wn_ref, bs_ref, bn_ref, o_ref):
    xb = x_ref[...]
    s = jnp.concatenate(
        [p_ref[0, 0] + p_ref[1, 0], p_ref[0, 1] + p_ref[1, 1]], axis=-1)
    ct = c_ref[0, :, 0] + c_ref[1, :, 0]
    mean = s / jnp.maximum(ct, 1.0)[:, None]
    nei = jnp.dot(mean, wn_ref[...], preferred_element_type=jnp.float32)
    nei = nei + bn_ref[...]
    sx = jnp.dot(xb, ws_ref[...], preferred_element_type=jnp.float32)
    sx = sx + bs_ref[...]
    o_ref[...] = jnp.maximum(sx + jnp.where(ct[:, None] > 0, nei, 0.0), 0.0)

  return pl.pallas_call(
      body,
      grid=(N // blk,),
      in_specs=[
          pl.BlockSpec((blk, D), lambda i: (i, 0)),
          pl.BlockSpec((NC, 2, blk, DH), lambda i: (0, 0, i, 0)),
          pl.BlockSpec((NC, blk, 16), lambda i: (0, i, 0)),
          pl.BlockSpec((D, D), lambda i: (0, 0)),
          pl.BlockSpec((D, D), lambda i: (0, 0)),
          pl.BlockSpec((1, D), lambda i: (0, 0)),
          pl.BlockSpec((1, D), lambda i: (0, 0)),
      ],
      out_specs=pl.BlockSpec((blk, D), lambda i: (i, 0)),
      out_shape=jax.ShapeDtypeStruct((N, D), jnp.float32),
  )(x, psum, pcnt, ws_t, wn_t, bs, bn)


def kernel(x, edge_index, W_self, b_self, W_nei, b_nei):
  dst = edge_index[0]
  src = edge_index[1]
  pad = E_PAD - E
  fill = jnp.arange(pad, dtype=jnp.int32)
  dst_p = jnp.concatenate([dst, N + fill % (N_PAD - N)])
  src_p = jnp.concatenate([src, fill % N])
  dst_r = dst_p.reshape(NW, CPT, CHUNK)
  src_r = src_p.reshape(NW, CPT, CHUNK)
  x0 = x[:, :DH]
  x1 = x[:, DH:]
  zrows = jnp.zeros((RPT, DH), jnp.float32)
  zcnt = jnp.zeros((RPT, 16), jnp.float32)
  ones_c = jnp.ones((CHUNK, 16), jnp.float32)
  psum, pcnt = _sc_segment_sum(x0, x1, dst_r, src_r, zrows, zcnt, ones_c)
  return _tc_combine(x, psum, pcnt, W_self.T, W_nei.T,
                     b_self[None, :], b_nei[None, :])


# speedup vs baseline: 3.1387x; 1.4568x over previous
import functools

import jax
import jax.numpy as jnp
from jax import lax
from jax.experimental import pallas as pl
from jax.experimental.pallas import tpu as pltpu
from jax.experimental.pallas import tpu_sc as plsc

N = 10000
D = 128
DH = D // 2
E = 320000

NC = 2
NS = 16
NW = NC * NS
CHUNK = 128
CPT = 80
NBUF = 2
EPT = CPT * CHUNK
E_PAD = EPT * NW
N_PAD = 10240
RPT = N_PAD // NS


def _sc_segment_sum(x0, x1, dst_r, src_r, zrows, zcnt, ones_c):
  mesh = plsc.VectorSubcoreMesh(core_axis_name="c", subcore_axis_name="s")

  @functools.partial(
      pl.kernel,
      out_type=[
          jax.ShapeDtypeStruct((NC, 2, N_PAD, DH), jnp.float32),
          jax.ShapeDtypeStruct((NC, N_PAD, 16), jnp.float32),
      ],
      mesh=mesh,
      compiler_params=pltpu.CompilerParams(use_tc_tiling_on_sc=False),
      scratch_types=[
          pltpu.VMEM((CPT, CHUNK), jnp.int32),
          pltpu.VMEM((CPT, CHUNK), jnp.int32),
          pltpu.VMEM((NBUF, CHUNK, DH), jnp.float32),
          pltpu.VMEM((CHUNK, 16), jnp.float32),
          pltpu.VMEM_SHARED((N_PAD, DH), jnp.float32),
          pltpu.VMEM_SHARED((N_PAD, 16), jnp.float32),
      ] + [pltpu.SemaphoreType.DMA] * NBUF,
  )
  def sc_kernel(x0_hbm, x1_hbm, dst_hbm, src_hbm, zr_hbm, zc_hbm, ones_hbm,
                sum_out, cnt_out,
                dst_v, src_v, rows_v, ones_v, acc_sh, cnt_sh, *sems):
    cid = lax.axis_index("c")
    sid = lax.axis_index("s")
    wid = sid * NC + cid
    r0 = sid * RPT

    pltpu.sync_copy(dst_hbm.at[wid], dst_v)
    pltpu.sync_copy(src_hbm.at[wid], src_v)
    pltpu.sync_copy(ones_hbm, ones_v)

    for h, xh_hbm in ((0, x0_hbm), (1, x1_hbm)):
      pltpu.sync_copy(zr_hbm, acc_sh.at[pl.ds(r0, RPT)])
      if h == 0:
        pltpu.sync_copy(zc_hbm, cnt_sh.at[pl.ds(r0, RPT)])
      plsc.subcore_barrier()

      for b in range(NBUF):
        pltpu.async_copy(xh_hbm.at[src_v.at[b]], rows_v.at[b], sems[b])

      def body(g, carry):
        for b in range(NBUF):
          j = g * NBUF + b
          pltpu.make_async_copy(xh_hbm.at[src_v.at[j]],
                                rows_v.at[b], sems[b]).wait()
          pltpu.sync_copy(rows_v.at[b], acc_sh.at[dst_v.at[j]], add=True)
          if h == 0:
            pltpu.sync_copy(ones_v, cnt_sh.at[dst_v.at[j]], add=True)
          jn = j + NBUF

          @pl.when(jn < CPT)
          def _():
            pltpu.async_copy(xh_hbm.at[src_v.at[jn]], rows_v.at[b], sems[b])

        return carry

      lax.fori_loop(0, CPT // NBUF, body, 0)
      plsc.subcore_barrier()

      pltpu.sync_copy(acc_sh.at[pl.ds(r0, RPT)],
                      sum_out.at[cid, h, pl.ds(r0, RPT)])
      if h == 0:
        pltpu.sync_copy(cnt_sh.at[pl.ds(r0, RPT)],
                        cnt_out.at[cid, pl.ds(r0, RPT)])
        plsc.subcore_barrier()

  return sc_kernel(x0, x1, dst_r, src_r, zrows, zcnt, ones_c)


def _tc_combine(x, psum, pcnt, ws_t, wn_t, bs, bn):
  blk = 1000

  def body(x_ref, p_ref, c_ref, ws_ref, wn_ref, bs_ref, bn_ref, o_ref):
    xb = x_ref[...]
    s = jnp.concatenate(
        [p_ref[0, 0] + p_ref[1, 0], p_ref[0, 1] + p_ref[1, 1]], axis=-1)
    ct = c_ref[0, :, 0] + c_ref[1, :, 0]
    mean = s / jnp.maximum(ct, 1.0)[:, None]
    nei = jnp.dot(mean, wn_ref[...], preferred_element_type=jnp.float32)
    nei = nei + bn_ref[...]
    sx = jnp.dot(xb, ws_ref[...], preferred_element_type=jnp.float32)
    sx = sx + bs_ref[...]
    o_ref[...] = jnp.maximum(sx + jnp.where(ct[:, None] > 0, nei, 0.0), 0.0)

  return pl.pallas_call(
      body,
      grid=(N // blk,),
      in_specs=[
          pl.BlockSpec((blk, D), lambda i: (i, 0)),
          pl.BlockSpec((NC, 2, blk, DH), lambda i: (0, 0, i, 0)),
          pl.BlockSpec((NC, blk, 16), lambda i: (0, i, 0)),
          pl.BlockSpec((D, D), lambda i: (0, 0)),
          pl.BlockSpec((D, D), lambda i: (0, 0)),
          pl.BlockSpec((1, D), lambda i: (0, 0)),
          pl.BlockSpec((1, D), lambda i: (0, 0)),
      ],
      out_specs=pl.BlockSpec((blk, D), lambda i: (i, 0)),
      out_shape=jax.ShapeDtypeStruct((N, D), jnp.float32),
  )(x, psum, pcnt, ws_t, wn_t, bs, bn)


def kernel(x, edge_index, W_self, b_self, W_nei, b_nei):
  dst = edge_index[0]
  src = edge_index[1]
  pad = E_PAD - E
  fill = jnp.arange(pad, dtype=jnp.int32)
  dst_p = jnp.concatenate([dst, N + fill % (N_PAD - N)])
  src_p = jnp.concatenate([src, fill % N])
  dst_r = dst_p.reshape(NW, CPT, CHUNK)
  src_r = src_p.reshape(NW, CPT, CHUNK)
  x0 = x[:, :DH]
  x1 = x[:, DH:]
  zrows = jnp.zeros((RPT, DH), jnp.float32)
  zcnt = jnp.zeros((RPT, 16), jnp.float32)
  ones_c = jnp.ones((CHUNK, 16), jnp.float32)
  psum, pcnt = _sc_segment_sum(x0, x1, dst_r, src_r, zrows, zcnt, ones_c)
  return _tc_combine(x, psum, pcnt, W_self.T, W_nei.T,
                     b_self[None, :], b_nei[None, :])


# speedup vs baseline: 3.5498x; 1.1310x over previous
import functools

import jax
import jax.numpy as jnp
from jax import lax
from jax.experimental import pallas as pl
from jax.experimental.pallas import tpu as pltpu
from jax.experimental.pallas import tpu_sc as plsc

N = 10000
D = 128
DH = D // 2
E = 320000

NC = 2
NS = 16
NW = NC * NS
CHUNK = 128
CPT = 80
NBUF = 5
PREF = 2
EPT = CPT * CHUNK
E_PAD = EPT * NW
N_PAD = 10240
RPT = N_PAD // NS


def _sc_segment_sum(x0, x1, dst_r, src_r, zrows, zcnt, ones_c):
  mesh = plsc.VectorSubcoreMesh(core_axis_name="c", subcore_axis_name="s")

  @functools.partial(
      pl.kernel,
      out_type=[
          jax.ShapeDtypeStruct((NC, 2, N_PAD, DH), jnp.float32),
          jax.ShapeDtypeStruct((NC, N_PAD, 16), jnp.float32),
      ],
      mesh=mesh,
      compiler_params=pltpu.CompilerParams(use_tc_tiling_on_sc=False),
      scratch_types=[
          pltpu.VMEM((CPT, CHUNK), jnp.int32),
          pltpu.VMEM((CPT, CHUNK), jnp.int32),
          pltpu.VMEM((NBUF, CHUNK, DH), jnp.float32),
          pltpu.VMEM((CHUNK, 16), jnp.float32),
          pltpu.VMEM_SHARED((N_PAD, DH), jnp.float32),
          pltpu.VMEM_SHARED((N_PAD, 16), jnp.float32),
      ] + [pltpu.SemaphoreType.DMA] * (3 * NBUF),
  )
  def sc_kernel(x0_hbm, x1_hbm, dst_hbm, src_hbm, zr_hbm, zc_hbm, ones_hbm,
                sum_out, cnt_out,
                dst_v, src_v, rows_v, ones_v, acc_sh, cnt_sh, *sems):
    g_sems = sems[:NBUF]
    s_sems = sems[NBUF:2 * NBUF]
    c_sems = sems[2 * NBUF:]
    cid = lax.axis_index("c")
    sid = lax.axis_index("s")
    wid = sid * NC + cid
    r0 = sid * RPT

    pltpu.sync_copy(dst_hbm.at[wid], dst_v)
    pltpu.sync_copy(src_hbm.at[wid], src_v)
    pltpu.sync_copy(ones_hbm, ones_v)

    for h, xh_hbm in ((0, x0_hbm), (1, x1_hbm)):
      pltpu.sync_copy(zr_hbm, acc_sh.at[pl.ds(r0, RPT)])
      if h == 0:
        pltpu.sync_copy(zc_hbm, cnt_sh.at[pl.ds(r0, RPT)])
      plsc.subcore_barrier()

      for b in range(PREF):
        pltpu.async_copy(xh_hbm.at[src_v.at[b]], rows_v.at[b], g_sems[b])

      def body(g, carry):
        for b in range(NBUF):
          j = g * NBUF + b
          jp = j + PREF
          bp = (b + PREF) % NBUF

          @pl.when(jp < CPT)
          def _():
            @pl.when(jp >= NBUF)
            def _():
              pltpu.make_async_copy(rows_v.at[bp], acc_sh.at[dst_v.at[0]],
                                    s_sems[bp]).wait()

            pltpu.async_copy(xh_hbm.at[src_v.at[jp]], rows_v.at[bp],
                             g_sems[bp])

          pltpu.make_async_copy(xh_hbm.at[src_v.at[j]],
                                rows_v.at[b], g_sems[b]).wait()
          pltpu.async_copy(rows_v.at[b], acc_sh.at[dst_v.at[j]], s_sems[b],
                           add=True)
          if h == 0:
            @pl.when(j >= NBUF)
            def _():
              pltpu.make_async_copy(ones_v, cnt_sh.at[dst_v.at[0]],
                                    c_sems[b]).wait()

            pltpu.async_copy(ones_v, cnt_sh.at[dst_v.at[j]], c_sems[b],
                             add=True)
        return carry

      lax.fori_loop(0, CPT // NBUF, body, 0)

      for b in range(NBUF):
        pltpu.make_async_copy(rows_v.at[b], acc_sh.at[dst_v.at[0]],
                              s_sems[b]).wait()
        if h == 0:
          pltpu.make_async_copy(ones_v, cnt_sh.at[dst_v.at[0]],
                                c_sems[b]).wait()
      plsc.subcore_barrier()

      pltpu.sync_copy(acc_sh.at[pl.ds(r0, RPT)],
                      sum_out.at[cid, h, pl.ds(r0, RPT)])
      if h == 0:
        pltpu.sync_copy(cnt_sh.at[pl.ds(r0, RPT)],
                        cnt_out.at[cid, pl.ds(r0, RPT)])
        plsc.subcore_barrier()

  return sc_kernel(x0, x1, dst_r, src_r, zrows, zcnt, ones_c)


def _tc_combine(x, psum, pcnt, ws_t, wn_t, bs, bn):
  blk = 1000

  def body(x_ref, p_ref, c_ref, ws_ref, wn_ref, bs_ref, bn_ref, o_ref):
    xb = x_ref[...]
    s = jnp.concatenate(
        [p_ref[0, 0] + p_ref[1, 0], p_ref[0, 1] + p_ref[1, 1]], axis=-1)
    ct = c_ref[0, :, 0] + c_ref[1, :, 0]
    mean = s / jnp.maximum(ct, 1.0)[:, None]
    nei = jnp.dot(mean, wn_ref[...], preferred_element_type=jnp.float32)
    nei = nei + bn_ref[...]
    sx = jnp.dot(xb, ws_ref[...], preferred_element_type=jnp.float32)
    sx = sx + bs_ref[...]
    o_ref[...] = jnp.maximum(sx + jnp.where(ct[:, None] > 0, nei, 0.0), 0.0)

  return pl.pallas_call(
      body,
      grid=(N // blk,),
      in_specs=[
          pl.BlockSpec((blk, D), lambda i: (i, 0)),
          pl.BlockSpec((NC, 2, blk, DH), lambda i: (0, 0, i, 0)),
          pl.BlockSpec((NC, blk, 16), lambda i: (0, i, 0)),
          pl.BlockSpec((D, D), lambda i: (0, 0)),
          pl.BlockSpec((D, D), lambda i: (0, 0)),
          pl.BlockSpec((1, D), lambda i: (0, 0)),
          pl.BlockSpec((1, D), lambda i: (0, 0)),
      ],
      out_specs=pl.BlockSpec((blk, D), lambda i: (i, 0)),
      out_shape=jax.ShapeDtypeStruct((N, D), jnp.float32),
  )(x, psum, pcnt, ws_t, wn_t, bs, bn)


def kernel(x, edge_index, W_self, b_self, W_nei, b_nei):
  dst = edge_index[0]
  src = edge_index[1]
  pad = E_PAD - E
  fill = jnp.arange(pad, dtype=jnp.int32)
  dst_p = jnp.concatenate([dst, N + fill % (N_PAD - N)])
  src_p = jnp.concatenate([src, fill % N])
  dst_r = dst_p.reshape(NW, CPT, CHUNK)
  src_r = src_p.reshape(NW, CPT, CHUNK)
  x0 = x[:, :DH]
  x1 = x[:, DH:]
  zrows = jnp.zeros((RPT, DH), jnp.float32)
  zcnt = jnp.zeros((RPT, 16), jnp.float32)
  ones_c = jnp.ones((CHUNK, 16), jnp.float32)
  psum, pcnt = _sc_segment_sum(x0, x1, dst_r, src_r, zrows, zcnt, ones_c)
  return _tc_combine(x, psum, pcnt, W_self.T, W_nei.T,
                     b_self[None, :], b_nei[None, :])


# speedup vs baseline: 3.5804x; 1.0086x over previous
import functools

import jax
import jax.numpy as jnp
from jax import lax
from jax.experimental import pallas as pl
from jax.experimental.pallas import tpu as pltpu
from jax.experimental.pallas import tpu_sc as plsc

N = 10000
D = 128
DH = D // 2
E = 320000

NC = 2
NS = 16
NW = NC * NS
CHUNK = 128
CPT = 80
NBUF = 5
PREF = 3
EPT = CPT * CHUNK
E_PAD = EPT * NW
N_PAD = 10240
RPT = N_PAD // NS


def _sc_segment_sum(x0, x1, dst_r, src_r, zrows, zcnt, ones_c):
  mesh = plsc.VectorSubcoreMesh(core_axis_name="c", subcore_axis_name="s")

  @functools.partial(
      pl.kernel,
      out_type=[
          jax.ShapeDtypeStruct((NC, 2, N_PAD, DH), jnp.float32),
          jax.ShapeDtypeStruct((NC, N_PAD, 16), jnp.float32),
      ],
      mesh=mesh,
      compiler_params=pltpu.CompilerParams(use_tc_tiling_on_sc=False),
      scratch_types=[
          pltpu.VMEM((CPT, CHUNK), jnp.int32),
          pltpu.VMEM((CPT, CHUNK), jnp.int32),
          pltpu.VMEM((NBUF, CHUNK, DH), jnp.float32),
          pltpu.VMEM((CHUNK, 16), jnp.float32),
          pltpu.VMEM_SHARED((N_PAD, DH), jnp.float32),
          pltpu.VMEM_SHARED((N_PAD, 16), jnp.float32),
      ] + [pltpu.SemaphoreType.DMA] * (3 * NBUF),
  )
  def sc_kernel(x0_hbm, x1_hbm, dst_hbm, src_hbm, zr_hbm, zc_hbm, ones_hbm,
                sum_out, cnt_out,
                dst_v, src_v, rows_v, ones_v, acc_sh, cnt_sh, *sems):
    g_sems = sems[:NBUF]
    s_sems = sems[NBUF:2 * NBUF]
    c_sems = sems[2 * NBUF:]
    cid = lax.axis_index("c")
    sid = lax.axis_index("s")
    wid = sid * NC + cid
    r0 = sid * RPT

    pltpu.sync_copy(dst_hbm.at[wid], dst_v)
    pltpu.sync_copy(src_hbm.at[wid], src_v)
    pltpu.sync_copy(ones_hbm, ones_v)

    for h, xh_hbm in ((0, x0_hbm), (1, x1_hbm)):
      pltpu.sync_copy(zr_hbm, acc_sh.at[pl.ds(r0, RPT)])
      if h == 0:
        pltpu.sync_copy(zc_hbm, cnt_sh.at[pl.ds(r0, RPT)])
      plsc.subcore_barrier()

      for b in range(PREF):
        pltpu.async_copy(xh_hbm.at[src_v.at[b]], rows_v.at[b], g_sems[b])

      def body(g, carry):
        for b in range(NBUF):
          j = g * NBUF + b
          jp = j + PREF
          bp = (b + PREF) % NBUF

          @pl.when(jp < CPT)
          def _():
            @pl.when(jp >= NBUF)
            def _():
              pltpu.make_async_copy(rows_v.at[bp], acc_sh.at[dst_v.at[0]],
                                    s_sems[bp]).wait()

            pltpu.async_copy(xh_hbm.at[src_v.at[jp]], rows_v.at[bp],
                             g_sems[bp])

          pltpu.make_async_copy(xh_hbm.at[src_v.at[j]],
                                rows_v.at[b], g_sems[b]).wait()
          pltpu.async_copy(rows_v.at[b], acc_sh.at[dst_v.at[j]], s_sems[b],
                           add=True)
          if h == 0:
            @pl.when(j >= NBUF)
            def _():
              pltpu.make_async_copy(ones_v, cnt_sh.at[dst_v.at[0]],
                                    c_sems[b]).wait()

            pltpu.async_copy(ones_v, cnt_sh.at[dst_v.at[j]], c_sems[b],
                             add=True)
        return carry

      lax.fori_loop(0, CPT // NBUF, body, 0)

      for b in range(NBUF):
        pltpu.make_async_copy(rows_v.at[b], acc_sh.at[dst_v.at[0]],
                              s_sems[b]).wait()
        if h == 0:
          pltpu.make_async_copy(ones_v, cnt_sh.at[dst_v.at[0]],
                                c_sems[b]).wait()
      plsc.subcore_barrier()

      pltpu.sync_copy(acc_sh.at[pl.ds(r0, RPT)],
                      sum_out.at[cid, h, pl.ds(r0, RPT)])
      if h == 0:
        pltpu.sync_copy(cnt_sh.at[pl.ds(r0, RPT)],
                        cnt_out.at[cid, pl.ds(r0, RPT)])
        plsc.subcore_barrier()

  return sc_kernel(x0, x1, dst_r, src_r, zrows, zcnt, ones_c)


def _tc_combine(x, psum, pcnt, ws_t, wn_t, bs, bn):
  blk = 1000

  def body(x_ref, p_ref, c_ref, ws_ref, wn_ref, bs_ref, bn_ref, o_ref):
    xb = x_ref[...]
    s = jnp.concatenate(
        [p_ref[0, 0] + p_ref[1, 0], p_ref[0, 1] + p_ref[1, 1]], axis=-1)
    ct = c_ref[0, :, 0] + c_ref[1, :, 0]
    mean = s / jnp.maximum(ct, 1.0)[:, None]
    nei = jnp.dot(mean, wn_ref[...], preferred_element_type=jnp.float32)
    nei = nei + bn_ref[...]
    sx = jnp.dot(xb, ws_ref[...], preferred_element_type=jnp.float32)
    sx = sx + bs_ref[...]
    o_ref[...] = jnp.maximum(sx + jnp.where(ct[:, None] > 0, nei, 0.0), 0.0)

  return pl.pallas_call(
      body,
      grid=(N // blk,),
      in_specs=[
          pl.BlockSpec((blk, D), lambda i: (i, 0)),
          pl.BlockSpec((NC, 2, blk, DH), lambda i: (0, 0, i, 0)),
          pl.BlockSpec((NC, blk, 16), lambda i: (0, i, 0)),
          pl.BlockSpec((D, D), lambda i: (0, 0)),
          pl.BlockSpec((D, D), lambda i: (0, 0)),
          pl.BlockSpec((1, D), lambda i: (0, 0)),
          pl.BlockSpec((1, D), lambda i: (0, 0)),
      ],
      out_specs=pl.BlockSpec((blk, D), lambda i: (i, 0)),
      out_shape=jax.ShapeDtypeStruct((N, D), jnp.float32),
  )(x, psum, pcnt, ws_t, wn_t, bs, bn)


def kernel(x, edge_index, W_self, b_self, W_nei, b_nei):
  dst = edge_index[0]
  src = edge_index[1]
  pad = E_PAD - E
  fill = jnp.arange(pad, dtype=jnp.int32)
  dst_p = jnp.concatenate([dst, N + fill % (N_PAD - N)])
  src_p = jnp.concatenate([src, fill % N])
  dst_r = dst_p.reshape(NW, CPT, CHUNK)
  src_r = src_p.reshape(NW, CPT, CHUNK)
  x0 = x[:, :DH]
  x1 = x[:, DH:]
  zrows = jnp.zeros((RPT, DH), jnp.float32)
  zcnt = jnp.zeros((RPT, 16), jnp.float32)
  ones_c = jnp.ones((CHUNK, 16), jnp.float32)
  psum, pcnt = _sc_segment_sum(x0, x1, dst_r, src_r, zrows, zcnt, ones_c)
  return _tc_combine(x, psum, pcnt, W_self.T, W_nei.T,
                     b_self[None, :], b_nei[None, :])
